# baseline (speedup 1.0000x reference)
import functools

import jax
import jax.numpy as jnp
from jax.experimental import pallas as pl


def _pad_rows(a, mult):
    n = a.shape[0]
    rem = (-n) % mult
    if rem:
        a = jnp.pad(a, ((0, rem),) + ((0, 0),) * (a.ndim - 1))
    return a


def _mm_kernel(x_ref, w_ref, b_ref, o_ref, *, act):
    acc = jnp.dot(x_ref[...], w_ref[...], preferred_element_type=jnp.float32)
    acc = acc + b_ref[...]
    if act == "leaky":
        acc = jnp.where(acc >= 0, acc, 0.01 * acc)
    o_ref[...] = acc


def _dense(x, w, b, act=None, block=512):
    n = x.shape[0]
    xp = _pad_rows(x, block)
    npad = xp.shape[0]
    k = w.shape[0]
    dout = w.shape[1]
    out = pl.pallas_call(
        functools.partial(_mm_kernel, act=act),
        grid=(npad // block,),
        in_specs=[
            pl.BlockSpec((block, k), lambda i: (i, 0)),
            pl.BlockSpec((k, dout), lambda i: (0, 0)),
            pl.BlockSpec((dout,), lambda i: (0,)),
        ],
        out_specs=pl.BlockSpec((block, dout), lambda i: (i, 0)),
        out_shape=jax.ShapeDtypeStruct((npad, dout), jnp.float32),
    )(xp, w, b)
    return out[:n]


def _layer(x, ea, src, dst, Wq, bq, Wk, bk, Wv, bv, Weg, Ws, bs):
    n = x.shape[0]
    c = Wq.shape[1]
    q = _dense(x, Wq, bq)
    k = _dense(x, Wk, bk)
    v = _dense(x, Wv, bv)
    qg = _dense(q, Weg.T, jnp.zeros((Weg.shape[0],), jnp.float32))
    alpha = (jnp.sum(q[dst] * k[src], axis=-1)
             + jnp.sum(qg[dst] * ea, axis=-1)) / jnp.sqrt(jnp.float32(c))
    amax = jax.ops.segment_max(alpha, dst, num_segments=n)
    amax = jnp.where(jnp.isfinite(amax), amax, 0.0)
    ex = jnp.exp(alpha - amax[dst])
    denom = jax.ops.segment_sum(ex, dst, num_segments=n)
    w = ex / jnp.maximum(denom[dst], 1e-16)
    aggv = jax.ops.segment_sum(w[:, None] * v[src], dst, num_segments=n)
    agge = jax.ops.segment_sum(w[:, None] * ea, dst, num_segments=n)
    out = aggv + _dense(agge, Weg, jnp.zeros((c,), jnp.float32)) + _dense(x, Ws, bs)
    return jnp.where(out >= 0, out, 0.01 * out)


def kernel(seq, node_s, edge_index, edge_s, batch, embed, Wn, bn, Wep, bep,
           Wq0, bq0, Wk0, bk0, Wv0, bv0, Weg0, Ws0, bs0,
           Wq1, bq1, Wk1, bk1, Wv1, bv1, Weg1, Ws1, bs1,
           Wq2, bq2, Wk2, bk2, Wv2, bv2, Weg2, Ws2, bs2,
           Wout, bout):
    src = edge_index[0].astype(jnp.int32)
    dst = edge_index[1].astype(jnp.int32)
    x = jnp.concatenate([embed[seq], node_s], axis=-1)
    x = _dense(x, Wn, bn)
    ea = _dense(edge_s, Wep, bep)
    x = _layer(x, ea, src, dst, Wq0, bq0, Wk0, bk0, Wv0, bv0, Weg0, Ws0, bs0)
    x = _layer(x, ea, src, dst, Wq1, bq1, Wk1, bk1, Wv1, bv1, Weg1, Ws1, bs1)
    x = _layer(x, ea, src, dst, Wq2, bq2, Wk2, bk2, Wv2, bv2, Weg2, Ws2, bs2)
    b32 = batch.astype(jnp.int32)
    cnt = jax.ops.segment_sum(jnp.ones((x.shape[0],), jnp.float32), b32, num_segments=32)
    pooled = jax.ops.segment_sum(x, b32, num_segments=32) / jnp.maximum(cnt, 1.0)[:, None]
    return _dense(pooled, Wout, bout)
